# baseline (speedup 1.0000x reference)
import functools

import jax
import jax.numpy as jnp
from jax import lax
from jax.experimental import pallas as pl
from jax.experimental.pallas import tpu as pltpu
from jax.experimental.pallas import tpu_sc as plsc

_VOCAB = 8192
_BATCH = 2
_SEQ = 2048
_B = _BATCH * _SEQ
_SPLIT = 4
_DS = _VOCAB // _SPLIT
_C = 8

_NC = 2
_NS = 16
_NW = _NC * _NS
_BPW = _B // _NW
_NQ = _BPW // _C


@functools.partial(
    pl.kernel,
    mesh=plsc.VectorSubcoreMesh(core_axis_name="c", subcore_axis_name="s"),
    out_type=jax.ShapeDtypeStruct((_B, _VOCAB), jnp.float32),
    scratch_types=[
        pltpu.VMEM((_BPW,), jnp.int32),
        pltpu.VMEM((_C, _DS), jnp.float32),
        pltpu.VMEM((_C, _DS), jnp.float32),
        pltpu.VMEM((_C, _DS), jnp.float32),
        pltpu.VMEM((_C, _DS), jnp.float32),
        pltpu.SemaphoreType.DMA,
        pltpu.SemaphoreType.DMA,
        pltpu.SemaphoreType.DMA,
        pltpu.SemaphoreType.DMA,
    ],
)
def _gather(
    tab_hbm, idx_hbm, out_hbm, idx_v,
    buf0, buf1, buf2, buf3, sem0, sem1, sem2, sem3,
):
    bufs = (buf0, buf1, buf2, buf3)
    sems = (sem0, sem1, sem2, sem3)
    wid = lax.axis_index("s") * _NC + lax.axis_index("c")
    base = wid * _BPW

    pltpu.sync_copy(idx_hbm.at[pl.ds(base, _BPW)], idx_v)

    def start_gather(q, h):
        pltpu.async_copy(
            tab_hbm.at[idx_v.at[pl.ds(q * _C, _C)], pl.ds(h * _DS, _DS)],
            bufs[h],
            sems[h],
        )

    def wait_gather(h):
        pltpu.make_async_copy(
            tab_hbm.at[idx_v.at[pl.ds(0, _C)], pl.ds(h * _DS, _DS)],
            bufs[h],
            sems[h],
        ).wait()

    for h in range(_SPLIT):
        start_gather(0, h)

    def outer(q, carry):
        for h in range(_SPLIT):
            wait_gather(h)
            pltpu.sync_copy(
                bufs[h],
                out_hbm.at[pl.ds(base + q * _C, _C), pl.ds(h * _DS, _DS)],
            )

            @pl.when(q + 1 < _NQ)
            def _():
                start_gather(q + 1, h)

        return carry

    lax.fori_loop(0, _NQ, outer, 0)


def kernel(index, targets, token_embedding_table):
    del targets
    idx = index.reshape(_B).astype(jnp.int32)
    out = _gather(token_embedding_table, idx)
    return out.reshape(_BATCH, _SEQ, _VOCAB)


# speedup vs baseline: 1.0054x; 1.0054x over previous
import functools

import jax
import jax.numpy as jnp
from jax import lax
from jax.experimental import pallas as pl
from jax.experimental.pallas import tpu as pltpu
from jax.experimental.pallas import tpu_sc as plsc

_VOCAB = 8192
_BATCH = 2
_SEQ = 2048
_B = _BATCH * _SEQ
_SPLIT = 4
_DS = _VOCAB // _SPLIT
_C = 8

_NC = 2
_NS = 16
_NW = _NC * _NS
_BPW = _B // _NW
_NQ = _BPW // _C


@functools.partial(
    pl.kernel,
    mesh=plsc.VectorSubcoreMesh(core_axis_name="c", subcore_axis_name="s"),
    out_type=jax.ShapeDtypeStruct((_B, _VOCAB), jnp.float32),
    scratch_types=[
        pltpu.VMEM((_BPW,), jnp.int32),
        pltpu.VMEM((_C, _DS), jnp.float32),
        pltpu.VMEM((_C, _DS), jnp.float32),
        pltpu.VMEM((_C, _DS), jnp.float32),
        pltpu.VMEM((_C, _DS), jnp.float32),
        pltpu.SemaphoreType.DMA,
        pltpu.SemaphoreType.DMA,
        pltpu.SemaphoreType.DMA,
        pltpu.SemaphoreType.DMA,
        pltpu.SemaphoreType.DMA,
        pltpu.SemaphoreType.DMA,
        pltpu.SemaphoreType.DMA,
        pltpu.SemaphoreType.DMA,
    ],
)
def _gather(
    tab_hbm, idx_hbm, out_hbm, idx_v,
    buf0, buf1, buf2, buf3,
    sem0, sem1, sem2, sem3, ssem0, ssem1, ssem2, ssem3,
):
    bufs = (buf0, buf1, buf2, buf3)
    sems = (sem0, sem1, sem2, sem3)
    ssems = (ssem0, ssem1, ssem2, ssem3)
    wid = lax.axis_index("s") * _NC + lax.axis_index("c")
    base = wid * _BPW

    pltpu.sync_copy(idx_hbm.at[pl.ds(base, _BPW)], idx_v)

    def start_gather(q, h):
        pltpu.async_copy(
            tab_hbm.at[idx_v.at[pl.ds(q * _C, _C)], pl.ds(h * _DS, _DS)],
            bufs[h],
            sems[h],
        )

    def wait_gather(h):
        pltpu.make_async_copy(
            tab_hbm.at[idx_v.at[pl.ds(0, _C)], pl.ds(h * _DS, _DS)],
            bufs[h],
            sems[h],
        ).wait()

    def start_store(q, h):
        pltpu.async_copy(
            bufs[h],
            out_hbm.at[pl.ds(base + q * _C, _C), pl.ds(h * _DS, _DS)],
            ssems[h],
        )

    def wait_store(h):
        pltpu.make_async_copy(
            bufs[h],
            out_hbm.at[pl.ds(base, _C), pl.ds(h * _DS, _DS)],
            ssems[h],
        ).wait()

    start_gather(0, 0)
    start_gather(0, 1)

    def outer(q, carry):
        for h in range(_SPLIT):
            h2 = (h + 2) % _SPLIT
            wait_gather(h)
            start_store(q, h)
            if h < 2:

                @pl.when(q >= 1)
                def _():
                    wait_store(h2)

                start_gather(q, h2)
            else:

                @pl.when(q + 1 < _NQ)
                def _():
                    wait_store(h2)
                    start_gather(q + 1, h2)

        return carry

    lax.fori_loop(0, _NQ, outer, 0)
    for h in range(_SPLIT):
        wait_store(h)


def kernel(index, targets, token_embedding_table):
    del targets
    idx = index.reshape(_B).astype(jnp.int32)
    out = _gather(token_embedding_table, idx)
    return out.reshape(_BATCH, _SEQ, _VOCAB)
